# Initial kernel scaffold; baseline (speedup 1.0000x reference)
import jax
import jax.numpy as jnp
from jax.experimental import pallas as pl


def kernel(x, pos_embed):
    raise NotImplementedError("write your pallas kernel here")


# speedup vs baseline: 11.9295x; 11.9295x over previous
import functools

import jax
import jax.numpy as jnp
from jax.experimental import pallas as pl


def _add_pe_kernel(x_ref, pe_ref, o_ref):
    o_ref[...] = x_ref[...] + pe_ref[...]


@functools.partial(jax.jit, static_argnames=())
def kernel(x, pos_embed):
    B, L, D = x.shape
    bB = next(b for b in (128, 64, 32, 16, 8, 4, 2, 1) if B % b == 0)
    return pl.pallas_call(
        _add_pe_kernel,
        grid=(B // bB,),
        in_specs=[
            pl.BlockSpec((bB, L, D), lambda i: (i, 0, 0)),
            pl.BlockSpec((L, D), lambda i: (0, 0)),
        ],
        out_specs=pl.BlockSpec((bB, L, D), lambda i: (i, 0, 0)),
        out_shape=jax.ShapeDtypeStruct((B, L, D), x.dtype),
    )(x, pos_embed)
